# initial kernel scaffold (unmeasured)
import jax
import jax.numpy as jnp
from jax import lax
from jax.experimental import pallas as pl
from jax.experimental.pallas import tpu as pltpu

P = 32
M = 2048
N = 2048
CH = M // P


def kernel(A, B):
    m, k = A.shape
    k2, n = B.shape

    def body(a_ref, b_ref, out_ref, rs_buf, rs_send, rs_recv, ag_send, ag_recv):
        my = lax.axis_index("i")
        left = lax.rem(my + P - 1, P)
        right = lax.rem(my + 1, P)

        barrier = pltpu.get_barrier_semaphore()
        for nbr in (left, right):
            pl.semaphore_signal(
                barrier, inc=1, device_id=(nbr,),
                device_id_type=pl.DeviceIdType.MESH,
            )
        pl.semaphore_wait(barrier, 2)

        out_ref[...] = jnp.dot(
            a_ref[...], b_ref[...], preferred_element_type=jnp.float32
        )

        for s in range(P - 1):
            send_c = lax.rem(my - s + P, P)
            recv_c = lax.rem(my - s - 1 + P, P)
            rdma = pltpu.make_async_remote_copy(
                src_ref=out_ref.at[pl.ds(send_c * CH, CH), :],
                dst_ref=rs_buf.at[s],
                send_sem=rs_send.at[s],
                recv_sem=rs_recv.at[s],
                device_id=(right,),
                device_id_type=pl.DeviceIdType.MESH,
            )
            rdma.start()
            rdma.wait()
            out_ref[pl.ds(recv_c * CH, CH), :] = (
                out_ref[pl.ds(recv_c * CH, CH), :] + rs_buf[s]
            )

        for s in range(P - 1):
            send_c = lax.rem(my + 1 - s + P, P)
            recv_c = lax.rem(my - s + P, P)
            send = pltpu.make_async_remote_copy(
                src_ref=out_ref.at[pl.ds(send_c * CH, CH), :],
                dst_ref=out_ref.at[pl.ds(send_c * CH, CH), :],
                send_sem=ag_send.at[s],
                recv_sem=ag_recv.at[s],
                device_id=(right,),
                device_id_type=pl.DeviceIdType.MESH,
            )
            send.start()
            recv = pltpu.make_async_remote_copy(
                src_ref=out_ref.at[pl.ds(recv_c * CH, CH), :],
                dst_ref=out_ref.at[pl.ds(recv_c * CH, CH), :],
                send_sem=ag_send.at[s],
                recv_sem=ag_recv.at[s],
                device_id=(right,),
                device_id_type=pl.DeviceIdType.MESH,
            )
            send.wait_send()
            recv.wait_recv()

    return pl.pallas_call(
        body,
        out_shape=jax.ShapeDtypeStruct((m, n), jnp.float32),
        in_specs=[
            pl.BlockSpec(memory_space=pltpu.VMEM),
            pl.BlockSpec(memory_space=pltpu.VMEM),
        ],
        out_specs=pl.BlockSpec(memory_space=pltpu.VMEM),
        scratch_shapes=[
            pltpu.VMEM((P - 1, CH, N), jnp.float32),
            pltpu.SemaphoreType.DMA((P - 1,)),
            pltpu.SemaphoreType.DMA((P - 1,)),
            pltpu.SemaphoreType.DMA((P - 1,)),
            pltpu.SemaphoreType.DMA((P - 1,)),
        ],
        compiler_params=pltpu.CompilerParams(collective_id=0),
    )(A, B)


# baseline (device time: 503266 ns/iter reference)
import jax
import jax.numpy as jnp
from jax import lax
from jax.experimental import pallas as pl
from jax.experimental.pallas import tpu as pltpu

P = 32
M = 2048
N = 2048
CH = M // P


def kernel(A, B):
    m, k = A.shape
    k2, n = B.shape

    def body(a_ref, b_ref, out_ref, rs_buf, rs_send, rs_recv, ag_send, ag_recv):
        my = lax.axis_index("i")
        left = lax.rem(my + P - 1, P)
        right = lax.rem(my + 1, P)

        barrier = pltpu.get_barrier_semaphore()
        for nbr in (left, right):
            pl.semaphore_signal(
                barrier, inc=1, device_id=(nbr,),
                device_id_type=pl.DeviceIdType.MESH,
            )
        pl.semaphore_wait(barrier, 2)

        out_ref[...] = jnp.dot(
            a_ref[...], b_ref[...], preferred_element_type=jnp.float32
        )

        for s in range(P - 1):
            send_c = lax.rem(my - s + P, P)
            recv_c = lax.rem(my - s - 1 + P, P)
            rdma = pltpu.make_async_remote_copy(
                src_ref=out_ref.at[pl.ds(send_c * CH, CH), :],
                dst_ref=rs_buf.at[s],
                send_sem=rs_send.at[s],
                recv_sem=rs_recv.at[s],
                device_id=(right,),
                device_id_type=pl.DeviceIdType.MESH,
            )
            rdma.start()
            rdma.wait()
            out_ref[pl.ds(recv_c * CH, CH), :] = (
                out_ref[pl.ds(recv_c * CH, CH), :] + rs_buf[s]
            )

        for s in range(P - 1):
            send_c = lax.rem(my + 1 - s + P, P)
            recv_c = lax.rem(my - s + P, P)
            send = pltpu.make_async_remote_copy(
                src_ref=out_ref.at[pl.ds(send_c * CH, CH), :],
                dst_ref=out_ref.at[pl.ds(send_c * CH, CH), :],
                send_sem=ag_send.at[s],
                recv_sem=ag_recv.at[s],
                device_id=(right,),
                device_id_type=pl.DeviceIdType.MESH,
            )
            send.start()
            recv = pltpu.make_async_remote_copy(
                src_ref=out_ref.at[pl.ds(recv_c * CH, CH), :],
                dst_ref=out_ref.at[pl.ds(recv_c * CH, CH), :],
                send_sem=ag_send.at[s],
                recv_sem=ag_recv.at[s],
                device_id=(right,),
                device_id_type=pl.DeviceIdType.MESH,
            )
            send.wait_send()
            recv.wait_recv()

    return pl.pallas_call(
        body,
        out_shape=jax.ShapeDtypeStruct((m, n), jnp.float32),
        in_specs=[
            pl.BlockSpec(memory_space=pltpu.VMEM),
            pl.BlockSpec(memory_space=pltpu.VMEM),
        ],
        out_specs=pl.BlockSpec(memory_space=pltpu.VMEM),
        scratch_shapes=[
            pltpu.VMEM((P - 1, CH, N), jnp.float32),
            pltpu.SemaphoreType.DMA((P - 1,)),
            pltpu.SemaphoreType.DMA((P - 1,)),
            pltpu.SemaphoreType.DMA((P - 1,)),
            pltpu.SemaphoreType.DMA((P - 1,)),
        ],
        compiler_params=pltpu.CompilerParams(
            collective_id=0,
            vmem_limit_bytes=100 * 1024 * 1024,
        ),
    )(A, B)


# device time: 475006 ns/iter; 1.0595x vs baseline; 1.0595x over previous
import jax
import jax.numpy as jnp
from jax import lax
from jax.experimental import pallas as pl
from jax.experimental.pallas import tpu as pltpu

P = 32
M = 2048
N = 2048
HN = N // 2
CH = M // P


def kernel(A, B):
    m, k = A.shape
    k2, n = B.shape

    def body(a_ref, b_ref, out_ref, rsp_buf, rsm_buf,
             rsp_send, rsp_recv, rsm_send, rsm_recv,
             agp_send, agp_recv, agm_send, agm_recv):
        my = lax.axis_index("i")
        left = lax.rem(my + P - 1, P)
        right = lax.rem(my + 1, P)

        def rows(c):
            return pl.ds(c * CH, CH)

        barrier = pltpu.get_barrier_semaphore()
        for nbr in (left, right):
            pl.semaphore_signal(
                barrier, inc=1, device_id=(nbr,),
                device_id_type=pl.DeviceIdType.MESH,
            )
        pl.semaphore_wait(barrier, 2)

        out_ref[...] = jnp.dot(
            a_ref[...], b_ref[...], preferred_element_type=jnp.float32
        )

        for s in range(P - 1):
            scp = lax.rem(my - s + P, P)
            rcp = lax.rem(my - s - 1 + P, P)
            scm = lax.rem(my + s, P)
            rcm = lax.rem(my + s + 1, P)
            sp = pltpu.make_async_remote_copy(
                src_ref=out_ref.at[rows(scp), pl.ds(0, HN)],
                dst_ref=rsp_buf.at[s],
                send_sem=rsp_send.at[s],
                recv_sem=rsp_recv.at[s],
                device_id=(right,),
                device_id_type=pl.DeviceIdType.MESH,
            )
            sm = pltpu.make_async_remote_copy(
                src_ref=out_ref.at[rows(scm), pl.ds(HN, HN)],
                dst_ref=rsm_buf.at[s],
                send_sem=rsm_send.at[s],
                recv_sem=rsm_recv.at[s],
                device_id=(left,),
                device_id_type=pl.DeviceIdType.MESH,
            )
            sp.start()
            sm.start()
            sp.wait()
            out_ref[rows(rcp), pl.ds(0, HN)] = (
                out_ref[rows(rcp), pl.ds(0, HN)] + rsp_buf[s]
            )
            sm.wait()
            out_ref[rows(rcm), pl.ds(HN, HN)] = (
                out_ref[rows(rcm), pl.ds(HN, HN)] + rsm_buf[s]
            )

        for s in range(P - 1):
            scp = lax.rem(my + 1 - s + P, P)
            rcp = lax.rem(my - s + P, P)
            scm = lax.rem(my - 1 + s + P, P)
            rcm = lax.rem(my + s, P)
            sp = pltpu.make_async_remote_copy(
                src_ref=out_ref.at[rows(scp), pl.ds(0, HN)],
                dst_ref=out_ref.at[rows(scp), pl.ds(0, HN)],
                send_sem=agp_send.at[s],
                recv_sem=agp_recv.at[s],
                device_id=(right,),
                device_id_type=pl.DeviceIdType.MESH,
            )
            sm = pltpu.make_async_remote_copy(
                src_ref=out_ref.at[rows(scm), pl.ds(HN, HN)],
                dst_ref=out_ref.at[rows(scm), pl.ds(HN, HN)],
                send_sem=agm_send.at[s],
                recv_sem=agm_recv.at[s],
                device_id=(left,),
                device_id_type=pl.DeviceIdType.MESH,
            )
            sp.start()
            sm.start()
            rp = pltpu.make_async_remote_copy(
                src_ref=out_ref.at[rows(rcp), pl.ds(0, HN)],
                dst_ref=out_ref.at[rows(rcp), pl.ds(0, HN)],
                send_sem=agp_send.at[s],
                recv_sem=agp_recv.at[s],
                device_id=(right,),
                device_id_type=pl.DeviceIdType.MESH,
            )
            rm = pltpu.make_async_remote_copy(
                src_ref=out_ref.at[rows(rcm), pl.ds(HN, HN)],
                dst_ref=out_ref.at[rows(rcm), pl.ds(HN, HN)],
                send_sem=agm_send.at[s],
                recv_sem=agm_recv.at[s],
                device_id=(left,),
                device_id_type=pl.DeviceIdType.MESH,
            )
            sp.wait_send()
            sm.wait_send()
            rp.wait_recv()
            rm.wait_recv()

    nsem = P - 1
    return pl.pallas_call(
        body,
        out_shape=jax.ShapeDtypeStruct((m, n), jnp.float32),
        in_specs=[
            pl.BlockSpec(memory_space=pltpu.VMEM),
            pl.BlockSpec(memory_space=pltpu.VMEM),
        ],
        out_specs=pl.BlockSpec(memory_space=pltpu.VMEM),
        scratch_shapes=[
            pltpu.VMEM((nsem, CH, HN), jnp.float32),
            pltpu.VMEM((nsem, CH, HN), jnp.float32),
            pltpu.SemaphoreType.DMA((nsem,)),
            pltpu.SemaphoreType.DMA((nsem,)),
            pltpu.SemaphoreType.DMA((nsem,)),
            pltpu.SemaphoreType.DMA((nsem,)),
            pltpu.SemaphoreType.DMA((nsem,)),
            pltpu.SemaphoreType.DMA((nsem,)),
            pltpu.SemaphoreType.DMA((nsem,)),
            pltpu.SemaphoreType.DMA((nsem,)),
        ],
        compiler_params=pltpu.CompilerParams(
            collective_id=0,
            vmem_limit_bytes=100 * 1024 * 1024,
        ),
    )(A, B)
